# Initial kernel scaffold; baseline (speedup 1.0000x reference)
import jax
import jax.numpy as jnp
from jax.experimental import pallas as pl


def kernel(x, target):
    raise NotImplementedError("write your pallas kernel here")


# speedup vs baseline: 5.9856x; 5.9856x over previous
import numpy as np
import jax
import jax.numpy as jnp
from jax.experimental import pallas as pl
from jax.experimental.pallas import tpu as pltpu

_SIZE = 32000
_N = 4096
_EPS = np.float32(1e-7)
_C = np.float32(0.1 / (_SIZE - 2))
_LOG_C = np.float32(np.log(np.float64(_C)))
_LOG_EPS = np.float32(np.log(np.float64(_EPS)))
_LOG_CONF = np.float32(np.log(np.float64(np.float32(0.9))))

_RB = 512
_CB = 3200
_NR = _N // _RB
_NC = _SIZE // _CB


def _body(x_ref, t_ref, o_ref, acc0, acc1, acct, y0s):
    i = pl.program_id(0)
    j = pl.program_id(1)

    x = x_ref[...]
    yt = jnp.minimum(jnp.maximum(x, _EPS), 1.0)
    s0 = jnp.sum(yt, axis=1, keepdims=True)
    s1 = jnp.sum(yt * jnp.log(yt), axis=1, keepdims=True)

    t = t_ref[...]
    cols = jax.lax.broadcasted_iota(jnp.int32, (_RB, _CB), 1) + j * _CB
    st = jnp.sum(jnp.where(cols == t, yt, 0.0), axis=1, keepdims=True)

    @pl.when(j == 0)
    def _init():
        acc0[...] = s0
        acc1[...] = s1
        acct[...] = st
        y0s[...] = yt[:, 0:1]

    @pl.when(j > 0)
    def _accum():
        acc0[...] += s0
        acc1[...] += s1
        acct[...] += st

    @pl.when(jnp.logical_and(i == 0, j == 0))
    def _init_out():
        o_ref[0, 0] = 0.0

    @pl.when(j == _NC - 1)
    def _epilogue():
        S0 = acc0[...]
        S1 = acc1[...]
        ytv = acct[...]
        y0 = y0s[...]
        is_pad = (t == 0)
        loss_np = S1 - ((S0 - y0 - ytv) * _LOG_C + y0 * _LOG_EPS
                        + ytv * _LOG_CONF)
        loss_p = S1 - S0 * _LOG_EPS
        loss = jnp.where(is_pad, loss_p, loss_np)
        o_ref[0, 0] += jnp.sum(loss) / np.float32(_N)


def _run(x, t2d, interpret=False):
    return pl.pallas_call(
        _body,
        grid=(_NR, _NC),
        in_specs=[
            pl.BlockSpec((_RB, _CB), lambda i, j: (i, j)),
            pl.BlockSpec((_RB, 1), lambda i, j: (i, 0)),
        ],
        out_specs=pl.BlockSpec((1, 1), lambda i, j: (0, 0),
                               memory_space=pltpu.SMEM),
        out_shape=jax.ShapeDtypeStruct((1, 1), jnp.float32),
        scratch_shapes=[
            pltpu.VMEM((_RB, 1), jnp.float32),
            pltpu.VMEM((_RB, 1), jnp.float32),
            pltpu.VMEM((_RB, 1), jnp.float32),
            pltpu.VMEM((_RB, 1), jnp.float32),
        ],
        compiler_params=pltpu.CompilerParams(
            dimension_semantics=("arbitrary", "arbitrary"),
        ),
        interpret=interpret,
    )(x, t2d)


def kernel(x, target):
    t2d = target.astype(jnp.int32).reshape(_N, 1)
    out = _run(x, t2d)
    return out.reshape(())
